# Initial kernel scaffold; baseline (speedup 1.0000x reference)
import jax
import jax.numpy as jnp
from jax.experimental import pallas as pl


def kernel(x, pos_edge_index, neg_edge_index, W1, b1, W2, b2):
    raise NotImplementedError("write your pallas kernel here")


# speedup vs baseline: 1.0575x; 1.0575x over previous
import jax
import jax.numpy as jnp
from jax.experimental import pallas as pl
from jax.experimental.pallas import tpu as pltpu


def _mm_body(x_ref, w_ref, o_ref):
    o_ref[...] = jnp.dot(x_ref[...], w_ref[...], preferred_element_type=jnp.float32)


def _matmul(x, w):
    return pl.pallas_call(
        _mm_body,
        out_shape=jax.ShapeDtypeStruct((x.shape[0], w.shape[1]), jnp.float32),
    )(x, w)


def kernel(x, pos_edge_index, neg_edge_index, W1, b1, W2, b2):
    N = x.shape[0]

    def conv(h_in, W, b):
        h = _matmul(h_in, W)
        loop = jnp.arange(N, dtype=pos_edge_index.dtype)
        src = jnp.concatenate([pos_edge_index[0], loop])
        dst = jnp.concatenate([pos_edge_index[1], loop])
        deg = jax.ops.segment_sum(jnp.ones_like(src, jnp.float32), dst, num_segments=N)
        dis = jnp.where(deg > 0, jax.lax.rsqrt(jnp.maximum(deg, 1e-12)), 0.0)
        norm = dis[src] * dis[dst]
        out = jax.ops.segment_sum(h[src] * norm[:, None], dst, num_segments=N)
        return out + b

    z = jax.nn.relu(conv(x, W1, b1))
    z = conv(z, W2, b2)
    ei = jnp.concatenate([pos_edge_index, neg_edge_index], axis=-1)
    return (z[ei[0]] * z[ei[1]]).sum(axis=-1)


# speedup vs baseline: 38.1279x; 36.0534x over previous
import functools

import jax
import jax.numpy as jnp
from jax import lax
from jax.experimental import pallas as pl
from jax.experimental.pallas import tpu as pltpu
from jax.experimental.pallas import tpu_sc as plsc

N = 10000
NP = 10240
E = 320000
E2 = 2 * E
D_IN = 128
H1 = 64
H2 = 32
NC = 2
NS = 16
L = 16
CH = 128
NPW = NP // NS

_MESH = plsc.VectorSubcoreMesh(
    core_axis_name="c", subcore_axis_name="s", num_cores=NC, num_subcores=NS)
_SC_PARAMS = pltpu.CompilerParams(
    use_tc_tiling_on_sc=False, needs_layout_passes=False)

_f32 = jnp.float32
_i32 = jnp.int32


def _fill(ref, rows, width, value):
    dw = width // L

    def body(i, _):
        ref[i // dw, pl.ds((i % dw) * L, L)] = jnp.full((L,), value, _f32)
        return 0

    lax.fori_loop(0, rows * dw, body, 0)


_N_CHUNK_CORE = E // NC // CH
_NJ_DEG = -(-_N_CHUNK_CORE // NS)


def _deg_body(dst_hbm, deg_out, didx, ones_v, slice_v, deg_sh, sem_i, sem_s):
    c = lax.axis_index("c")
    s = lax.axis_index("s")
    _fill(ones_v, 1, CH, 1.0)
    _fill(slice_v, 1, NPW, 0.0)
    pltpu.sync_copy(slice_v.at[0], deg_sh.at[pl.ds(s * NPW, NPW)])
    plsc.subcore_barrier()

    ebase = c * _N_CHUNK_CORE * CH
    K = 8
    nblk = -(-_NJ_DEG // K)

    def blk(b, _):
        loads = []
        for k in range(K):
            cid = s + (b * K + k) * NS
            valid = cid < _N_CHUNK_CORE
            base = ebase + cid * CH

            @pl.when(valid)
            def _(k=k, base=base):
                loads.append(
                    pltpu.async_copy(dst_hbm.at[pl.ds(base, CH)], didx.at[k], sem_i))
        for k in range(K):
            cid = s + (b * K + k) * NS
            valid = cid < _N_CHUNK_CORE
            base = ebase + cid * CH

            @pl.when(valid)
            def _(k=k, base=base):
                pltpu.make_async_copy(
                    dst_hbm.at[pl.ds(base, CH)], didx.at[k], sem_i).wait()
                pltpu.async_copy(
                    ones_v.at[0], deg_sh.at[didx.at[k]], sem_s, add=True)
        for k in range(K):
            cid = s + (b * K + k) * NS
            valid = cid < _N_CHUNK_CORE

            @pl.when(valid)
            def _(k=k):
                pltpu.make_async_copy(
                    ones_v.at[0], deg_sh.at[didx.at[k]], sem_s).wait()
        return 0

    lax.fori_loop(0, nblk, blk, 0)
    plsc.subcore_barrier()
    pltpu.sync_copy(deg_sh.at[pl.ds(s * NPW, NPW)],
                    deg_out.at[c, pl.ds(s * NPW, NPW)])


_deg_call = functools.partial(
    pl.kernel,
    out_type=jax.ShapeDtypeStruct((NC, NP), _f32),
    mesh=_MESH,
    compiler_params=_SC_PARAMS,
    scratch_types=[
        pltpu.VMEM((8, CH), _i32),
        pltpu.VMEM((1, CH), _f32),
        pltpu.VMEM((1, NPW), _f32),
        pltpu.VMEM_SHARED((NP,), _f32),
        pltpu.SemaphoreType.DMA,
        pltpu.SemaphoreType.DMA,
    ],
)(_deg_body)


def _make_scatter(d):
    K = 8
    nblk = -(-_NJ_DEG // K)

    def body(src_hbm, dst_hbm, hs_hbm, acc_out,
             sidx, didx, rows, zrow, acc_sh, sem_i, sem_g, sem_s):
        c = lax.axis_index("c")
        s = lax.axis_index("s")
        _fill(zrow, CH, d, 0.0)
        for t in range(NPW // CH):
            pltpu.sync_copy(zrow, acc_sh.at[pl.ds(s * NPW + t * CH, CH)])
        plsc.subcore_barrier()

        ebase = c * _N_CHUNK_CORE * CH

        def blk(b, _):
            for k in range(K):
                cid = s + (b * K + k) * NS
                valid = cid < _N_CHUNK_CORE
                base = ebase + cid * CH

                @pl.when(valid)
                def _(k=k, base=base):
                    pltpu.async_copy(src_hbm.at[pl.ds(base, CH)], sidx.at[k], sem_i)
                    pltpu.async_copy(dst_hbm.at[pl.ds(base, CH)], didx.at[k], sem_i)
            for k in range(K):
                cid = s + (b * K + k) * NS
                valid = cid < _N_CHUNK_CORE
                base = ebase + cid * CH

                @pl.when(valid)
                def _(k=k, base=base):
                    pltpu.make_async_copy(
                        src_hbm.at[pl.ds(base, CH)], sidx.at[k], sem_i).wait()
                    pltpu.make_async_copy(
                        dst_hbm.at[pl.ds(base, CH)], didx.at[k], sem_i).wait()
                    pltpu.async_copy(hs_hbm.at[sidx.at[k]], rows.at[k], sem_g)
            for k in range(K):
                cid = s + (b * K + k) * NS
                valid = cid < _N_CHUNK_CORE

                @pl.when(valid)
                def _(k=k):
                    pltpu.make_async_copy(
                        hs_hbm.at[sidx.at[k]], rows.at[k], sem_g).wait()
                    pltpu.async_copy(
                        rows.at[k], acc_sh.at[didx.at[k]], sem_s, add=True)
            for k in range(K):
                cid = s + (b * K + k) * NS
                valid = cid < _N_CHUNK_CORE

                @pl.when(valid)
                def _(k=k):
                    pltpu.make_async_copy(
                        rows.at[k], acc_sh.at[didx.at[k]], sem_s).wait()
            return 0

        lax.fori_loop(0, nblk, blk, 0)
        plsc.subcore_barrier()
        pltpu.sync_copy(acc_sh.at[pl.ds(s * NPW, NPW)],
                        acc_out.at[c, pl.ds(s * NPW, NPW)])

    return pl.kernel(
        body,
        out_type=jax.ShapeDtypeStruct((NC, NP, d), _f32),
        mesh=_MESH,
        compiler_params=_SC_PARAMS,
        scratch_types=[
            pltpu.VMEM((K, CH), _i32),
            pltpu.VMEM((K, CH), _i32),
            pltpu.VMEM((K, CH, d), _f32),
            pltpu.VMEM((CH, d), _f32),
            pltpu.VMEM_SHARED((NP, d), _f32),
            pltpu.SemaphoreType.DMA,
            pltpu.SemaphoreType.DMA,
            pltpu.SemaphoreType.DMA,
        ],
    )


_scatter64 = _make_scatter(H1)
_scatter32 = _make_scatter(H2)


_N_CHUNK_DEC = E2 // CH
_NJ_DEC = -(-_N_CHUNK_DEC // (NC * NS))


def _dec_body(src_hbm, dst_hbm, z_hbm, out_hbm,
              sidx, didx, rows_s, rows_d, outv, sem_i, sem_g, sem_o):
    c = lax.axis_index("c")
    s = lax.axis_index("s")
    w = s * NC + c
    K = 4
    nblk = -(-_NJ_DEC // K)

    def blk(b, _):
        for k in range(K):
            cid = w + (b * K + k) * (NC * NS)
            valid = cid < _N_CHUNK_DEC
            base = cid * CH

            @pl.when(valid)
            def _(k=k, base=base):
                pltpu.async_copy(src_hbm.at[pl.ds(base, CH)], sidx.at[k], sem_i)
                pltpu.async_copy(dst_hbm.at[pl.ds(base, CH)], didx.at[k], sem_i)
        for k in range(K):
            cid = w + (b * K + k) * (NC * NS)
            valid = cid < _N_CHUNK_DEC
            base = cid * CH

            @pl.when(valid)
            def _(k=k, base=base):
                pltpu.make_async_copy(
                    src_hbm.at[pl.ds(base, CH)], sidx.at[k], sem_i).wait()
                pltpu.make_async_copy(
                    dst_hbm.at[pl.ds(base, CH)], didx.at[k], sem_i).wait()
                pltpu.async_copy(z_hbm.at[sidx.at[k]], rows_s.at[k], sem_g)
                pltpu.async_copy(z_hbm.at[didx.at[k]], rows_d.at[k], sem_g)
        for k in range(K):
            cid = w + (b * K + k) * (NC * NS)
            valid = cid < _N_CHUNK_DEC
            base = cid * CH

            @pl.when(valid)
            def _(k=k, base=base):
                pltpu.make_async_copy(
                    z_hbm.at[sidx.at[k]], rows_s.at[k], sem_g).wait()
                pltpu.make_async_copy(
                    z_hbm.at[didx.at[k]], rows_d.at[k], sem_g).wait()

                def gbody(g, _):
                    lanes = lax.iota(_i32, L)
                    v = jnp.zeros((L,), _f32)
                    for u in range(L):
                        e = g * L + u
                        a = (rows_s[k, e, pl.ds(0, L)] * rows_d[k, e, pl.ds(0, L)]
                             + rows_s[k, e, pl.ds(L, L)] * rows_d[k, e, pl.ds(L, L)])
                        v = jnp.where(lanes == u, jnp.sum(a), v)
                    outv[k, pl.ds(g * L, L)] = v
                    return 0

                lax.fori_loop(0, CH // L, gbody, 0)
                pltpu.async_copy(outv.at[k], out_hbm.at[pl.ds(base, CH)], sem_o)
        for k in range(K):
            cid = w + (b * K + k) * (NC * NS)
            valid = cid < _N_CHUNK_DEC
            base = cid * CH

            @pl.when(valid)
            def _(k=k, base=base):
                pltpu.make_async_copy(
                    outv.at[k], out_hbm.at[pl.ds(base, CH)], sem_o).wait()
        return 0

    lax.fori_loop(0, nblk, blk, 0)


_dec_call = pl.kernel(
    _dec_body,
    out_type=jax.ShapeDtypeStruct((E2,), _f32),
    mesh=_MESH,
    compiler_params=_SC_PARAMS,
    scratch_types=[
        pltpu.VMEM((4, CH), _i32),
        pltpu.VMEM((4, CH), _i32),
        pltpu.VMEM((4, CH, H2), _f32),
        pltpu.VMEM((4, CH, H2), _f32),
        pltpu.VMEM((4, CH), _f32),
        pltpu.SemaphoreType.DMA,
        pltpu.SemaphoreType.DMA,
        pltpu.SemaphoreType.DMA,
    ],
)


_BLK = 1024
_GRID = NP // _BLK


def _enc1_body(x_ref, w_ref, degt_ref, hs_ref, dis_ref):
    d2 = degt_ref[...]
    deg = d2[:, 0:1] + d2[:, 1:2] + 1.0
    dis = lax.rsqrt(deg)
    h = jnp.dot(x_ref[...], w_ref[...], preferred_element_type=_f32)
    hs_ref[...] = h * dis
    dis_ref[...] = dis


def _enc1_call(xp, W1, degt):
    return pl.pallas_call(
        _enc1_body,
        grid=(_GRID,),
        in_specs=[
            pl.BlockSpec((_BLK, D_IN), lambda i: (i, 0)),
            pl.BlockSpec((D_IN, H1), lambda i: (0, 0)),
            pl.BlockSpec((_BLK, NC), lambda i: (i, 0)),
        ],
        out_specs=[
            pl.BlockSpec((_BLK, H1), lambda i: (i, 0)),
            pl.BlockSpec((_BLK, 1), lambda i: (i, 0)),
        ],
        out_shape=[
            jax.ShapeDtypeStruct((NP, H1), _f32),
            jax.ShapeDtypeStruct((NP, 1), _f32),
        ],
    )(xp, W1, degt)


def _mid_body(a0_ref, a1_ref, hs_ref, dis_ref, b_ref, w_ref, o_ref):
    dis = dis_ref[...]
    z1 = jnp.maximum(
        (a0_ref[...] + a1_ref[...] + hs_ref[...]) * dis + b_ref[...], 0.0)
    o_ref[...] = jnp.dot(z1, w_ref[...], preferred_element_type=_f32) * dis


def _mid_call(a0, a1, hs1, dis, b1, W2):
    return pl.pallas_call(
        _mid_body,
        grid=(_GRID,),
        in_specs=[
            pl.BlockSpec((_BLK, H1), lambda i: (i, 0)),
            pl.BlockSpec((_BLK, H1), lambda i: (i, 0)),
            pl.BlockSpec((_BLK, H1), lambda i: (i, 0)),
            pl.BlockSpec((_BLK, 1), lambda i: (i, 0)),
            pl.BlockSpec((1, H1), lambda i: (0, 0)),
            pl.BlockSpec((H1, H2), lambda i: (0, 0)),
        ],
        out_specs=pl.BlockSpec((_BLK, H2), lambda i: (i, 0)),
        out_shape=jax.ShapeDtypeStruct((NP, H2), _f32),
    )(a0, a1, hs1, dis, b1, W2)


def _fin_body(a0_ref, a1_ref, hs_ref, dis_ref, b_ref, o_ref):
    o_ref[...] = ((a0_ref[...] + a1_ref[...] + hs_ref[...]) * dis_ref[...]
                  + b_ref[...])


def _fin_call(a0, a1, hs2, dis, b2):
    return pl.pallas_call(
        _fin_body,
        grid=(_GRID,),
        in_specs=[
            pl.BlockSpec((_BLK, H2), lambda i: (i, 0)),
            pl.BlockSpec((_BLK, H2), lambda i: (i, 0)),
            pl.BlockSpec((_BLK, H2), lambda i: (i, 0)),
            pl.BlockSpec((_BLK, 1), lambda i: (i, 0)),
            pl.BlockSpec((1, H2), lambda i: (0, 0)),
        ],
        out_specs=pl.BlockSpec((_BLK, H2), lambda i: (i, 0)),
        out_shape=jax.ShapeDtypeStruct((NP, H2), _f32),
    )(a0, a1, hs2, dis, b2)


def kernel(x, pos_edge_index, neg_edge_index, W1, b1, W2, b2):
    src = pos_edge_index[0].astype(_i32)
    dst = pos_edge_index[1].astype(_i32)
    xp = jnp.pad(x, ((0, NP - N), (0, 0)))

    degp = _deg_call(dst)
    degt = degp.T
    hs1, dis = _enc1_call(xp, W1, degt)
    acc1 = _scatter64(src, dst, hs1)
    hs2 = _mid_call(acc1[0], acc1[1], hs1, dis, b1.reshape(1, H1), W2)
    acc2 = _scatter32(src, dst, hs2)
    z2 = _fin_call(acc2[0], acc2[1], hs2, dis, b2.reshape(1, H2))

    s2 = jnp.concatenate([src, neg_edge_index[0].astype(_i32)])
    d2 = jnp.concatenate([dst, neg_edge_index[1].astype(_i32)])
    return _dec_call(s2, d2, z2)


# speedup vs baseline: 39.9802x; 1.0486x over previous
import functools

import jax
import jax.numpy as jnp
from jax import lax
from jax.experimental import pallas as pl
from jax.experimental.pallas import tpu as pltpu
from jax.experimental.pallas import tpu_sc as plsc

N = 10000
NP = 10240
E = 320000
E2 = 2 * E
D_IN = 128
H1 = 64
H2 = 32
NC = 2
NS = 16
L = 16
CH = 128
NPW = NP // NS

_MESH = plsc.VectorSubcoreMesh(
    core_axis_name="c", subcore_axis_name="s", num_cores=NC, num_subcores=NS)
_SC_PARAMS = pltpu.CompilerParams(
    use_tc_tiling_on_sc=False, needs_layout_passes=False)

_f32 = jnp.float32
_i32 = jnp.int32


def _fill(ref, rows, width, value):
    dw = width // L

    def body(i, _):
        ref[i // dw, pl.ds((i % dw) * L, L)] = jnp.full((L,), value, _f32)
        return 0

    lax.fori_loop(0, rows * dw, body, 0)


_N_CHUNK_CORE = E // NC // CH
_NJ_DEG = -(-_N_CHUNK_CORE // NS)


def _deg_body(dst_hbm, deg_out, didx, ones_v, slice_v, deg_sh, sem_i, sem_s):
    c = lax.axis_index("c")
    s = lax.axis_index("s")
    _fill(ones_v, 1, CH, 1.0)
    _fill(slice_v, 1, NPW, 0.0)
    pltpu.sync_copy(slice_v.at[0], deg_sh.at[pl.ds(s * NPW, NPW)])
    plsc.subcore_barrier()

    ebase = c * _N_CHUNK_CORE * CH
    K = 8
    nblk = -(-_NJ_DEG // K)

    def blk(b, _):
        loads = []
        for k in range(K):
            cid = s + (b * K + k) * NS
            valid = cid < _N_CHUNK_CORE
            base = ebase + cid * CH

            @pl.when(valid)
            def _(k=k, base=base):
                loads.append(
                    pltpu.async_copy(dst_hbm.at[pl.ds(base, CH)], didx.at[k], sem_i))
        for k in range(K):
            cid = s + (b * K + k) * NS
            valid = cid < _N_CHUNK_CORE
            base = ebase + cid * CH

            @pl.when(valid)
            def _(k=k, base=base):
                pltpu.make_async_copy(
                    dst_hbm.at[pl.ds(base, CH)], didx.at[k], sem_i).wait()
                pltpu.async_copy(
                    ones_v.at[0], deg_sh.at[didx.at[k]], sem_s, add=True)
        for k in range(K):
            cid = s + (b * K + k) * NS
            valid = cid < _N_CHUNK_CORE

            @pl.when(valid)
            def _(k=k):
                pltpu.make_async_copy(
                    ones_v.at[0], deg_sh.at[didx.at[k]], sem_s).wait()
        return 0

    lax.fori_loop(0, nblk, blk, 0)
    plsc.subcore_barrier()
    pltpu.sync_copy(deg_sh.at[pl.ds(s * NPW, NPW)],
                    deg_out.at[c, pl.ds(s * NPW, NPW)])


_deg_call = functools.partial(
    pl.kernel,
    out_type=jax.ShapeDtypeStruct((NC, NP), _f32),
    mesh=_MESH,
    compiler_params=_SC_PARAMS,
    scratch_types=[
        pltpu.VMEM((8, CH), _i32),
        pltpu.VMEM((1, CH), _f32),
        pltpu.VMEM((1, NPW), _f32),
        pltpu.VMEM_SHARED((NP,), _f32),
        pltpu.SemaphoreType.DMA,
        pltpu.SemaphoreType.DMA,
    ],
)(_deg_body)


def _make_scatter(d):
    K = 8
    nblk = -(-_NJ_DEG // K)

    def body(src_hbm, dst_hbm, hs_hbm, acc_out,
             sidx, didx, rows, zrow, acc_sh, sem_i, sem_g, sem_s):
        c = lax.axis_index("c")
        s = lax.axis_index("s")
        _fill(zrow, CH, d, 0.0)
        for t in range(NPW // CH):
            pltpu.sync_copy(zrow, acc_sh.at[pl.ds(s * NPW + t * CH, CH)])
        plsc.subcore_barrier()

        ebase = c * _N_CHUNK_CORE * CH

        def blk(b, _):
            for k in range(K):
                cid = s + (b * K + k) * NS
                valid = cid < _N_CHUNK_CORE
                base = ebase + cid * CH

                @pl.when(valid)
                def _(k=k, base=base):
                    pltpu.async_copy(src_hbm.at[pl.ds(base, CH)], sidx.at[k], sem_i)
                    pltpu.async_copy(dst_hbm.at[pl.ds(base, CH)], didx.at[k], sem_i)
            for k in range(K):
                cid = s + (b * K + k) * NS
                valid = cid < _N_CHUNK_CORE
                base = ebase + cid * CH

                @pl.when(valid)
                def _(k=k, base=base):
                    pltpu.make_async_copy(
                        src_hbm.at[pl.ds(base, CH)], sidx.at[k], sem_i).wait()
                    pltpu.make_async_copy(
                        dst_hbm.at[pl.ds(base, CH)], didx.at[k], sem_i).wait()
                    pltpu.async_copy(hs_hbm.at[sidx.at[k]], rows.at[k], sem_g)
            for k in range(K):
                cid = s + (b * K + k) * NS
                valid = cid < _N_CHUNK_CORE

                @pl.when(valid)
                def _(k=k):
                    pltpu.make_async_copy(
                        hs_hbm.at[sidx.at[k]], rows.at[k], sem_g).wait()
                    pltpu.async_copy(
                        rows.at[k], acc_sh.at[didx.at[k]], sem_s, add=True)
            for k in range(K):
                cid = s + (b * K + k) * NS
                valid = cid < _N_CHUNK_CORE

                @pl.when(valid)
                def _(k=k):
                    pltpu.make_async_copy(
                        rows.at[k], acc_sh.at[didx.at[k]], sem_s).wait()
            return 0

        lax.fori_loop(0, nblk, blk, 0)
        plsc.subcore_barrier()
        pltpu.sync_copy(acc_sh.at[pl.ds(s * NPW, NPW)],
                        acc_out.at[c, pl.ds(s * NPW, NPW)])

    return pl.kernel(
        body,
        out_type=jax.ShapeDtypeStruct((NC, NP, d), _f32),
        mesh=_MESH,
        compiler_params=_SC_PARAMS,
        scratch_types=[
            pltpu.VMEM((K, CH), _i32),
            pltpu.VMEM((K, CH), _i32),
            pltpu.VMEM((K, CH, d), _f32),
            pltpu.VMEM((CH, d), _f32),
            pltpu.VMEM_SHARED((NP, d), _f32),
            pltpu.SemaphoreType.DMA,
            pltpu.SemaphoreType.DMA,
            pltpu.SemaphoreType.DMA,
        ],
    )


_scatter64 = _make_scatter(H1)
_scatter32 = _make_scatter(H2)


_N_CHUNK_DEC = E2 // CH
_NJ_DEC = -(-_N_CHUNK_DEC // (NC * NS))


def _dec_body(src_hbm, dst_hbm, z_hbm, out_hbm,
              sidx, didx, rows_s, rows_d, outv, sem_i, sem_g, sem_o):
    c = lax.axis_index("c")
    s = lax.axis_index("s")
    w = s * NC + c
    K = 8
    nblk = -(-_NJ_DEC // K)

    def blk(b, _):
        for k in range(K):
            cid = w + (b * K + k) * (NC * NS)
            valid = cid < _N_CHUNK_DEC
            base = cid * CH

            @pl.when(valid)
            def _(k=k, base=base):
                pltpu.async_copy(src_hbm.at[pl.ds(base, CH)], sidx.at[k], sem_i)
                pltpu.async_copy(dst_hbm.at[pl.ds(base, CH)], didx.at[k], sem_i)
        for k in range(K):
            cid = w + (b * K + k) * (NC * NS)
            valid = cid < _N_CHUNK_DEC
            base = cid * CH

            @pl.when(valid)
            def _(k=k, base=base):
                pltpu.make_async_copy(
                    src_hbm.at[pl.ds(base, CH)], sidx.at[k], sem_i).wait()
                pltpu.make_async_copy(
                    dst_hbm.at[pl.ds(base, CH)], didx.at[k], sem_i).wait()
                pltpu.async_copy(z_hbm.at[sidx.at[k]], rows_s.at[k], sem_g)
                pltpu.async_copy(z_hbm.at[didx.at[k]], rows_d.at[k], sem_g)
        for k in range(K):
            cid = w + (b * K + k) * (NC * NS)
            valid = cid < _N_CHUNK_DEC
            base = cid * CH

            @pl.when(valid)
            def _(k=k, base=base):
                pltpu.make_async_copy(
                    z_hbm.at[sidx.at[k]], rows_s.at[k], sem_g).wait()
                pltpu.make_async_copy(
                    z_hbm.at[didx.at[k]], rows_d.at[k], sem_g).wait()

                def gbody(g, _):
                    lanes = lax.iota(_i32, L)
                    v = jnp.zeros((L,), _f32)
                    for u in range(L):
                        e = g * L + u
                        sa, sb = plsc.unpack(
                            rows_s[k, e, :], format=plsc.PackFormat.INTERLEAVED)
                        da, db = plsc.unpack(
                            rows_d[k, e, :], format=plsc.PackFormat.INTERLEAVED)
                        a = sa * da + sb * db
                        v = jnp.where(lanes == u, jnp.sum(a), v)
                    outv[k, pl.ds(g * L, L)] = v
                    return 0

                lax.fori_loop(0, CH // L, gbody, 0)
                pltpu.async_copy(outv.at[k], out_hbm.at[pl.ds(base, CH)], sem_o)
        for k in range(K):
            cid = w + (b * K + k) * (NC * NS)
            valid = cid < _N_CHUNK_DEC
            base = cid * CH

            @pl.when(valid)
            def _(k=k, base=base):
                pltpu.make_async_copy(
                    outv.at[k], out_hbm.at[pl.ds(base, CH)], sem_o).wait()
        return 0

    lax.fori_loop(0, nblk, blk, 0)


_dec_call = pl.kernel(
    _dec_body,
    out_type=jax.ShapeDtypeStruct((E2,), _f32),
    mesh=_MESH,
    compiler_params=_SC_PARAMS,
    scratch_types=[
        pltpu.VMEM((8, CH), _i32),
        pltpu.VMEM((8, CH), _i32),
        pltpu.VMEM((8, CH, H2), jnp.bfloat16),
        pltpu.VMEM((8, CH, H2), jnp.bfloat16),
        pltpu.VMEM((8, CH), _f32),
        pltpu.SemaphoreType.DMA,
        pltpu.SemaphoreType.DMA,
        pltpu.SemaphoreType.DMA,
    ],
)


_BLK = 1024
_GRID = NP // _BLK


def _enc1_body(x_ref, w_ref, degt_ref, hs_ref, dis_ref):
    d2 = degt_ref[...]
    deg = d2[:, 0:1] + d2[:, 1:2] + 1.0
    dis = lax.rsqrt(deg)
    h = jnp.dot(x_ref[...], w_ref[...], preferred_element_type=_f32)
    hs_ref[...] = h * dis
    dis_ref[...] = dis


def _enc1_call(xp, W1, degt):
    return pl.pallas_call(
        _enc1_body,
        grid=(_GRID,),
        in_specs=[
            pl.BlockSpec((_BLK, D_IN), lambda i: (i, 0)),
            pl.BlockSpec((D_IN, H1), lambda i: (0, 0)),
            pl.BlockSpec((_BLK, NC), lambda i: (i, 0)),
        ],
        out_specs=[
            pl.BlockSpec((_BLK, H1), lambda i: (i, 0)),
            pl.BlockSpec((_BLK, 1), lambda i: (i, 0)),
        ],
        out_shape=[
            jax.ShapeDtypeStruct((NP, H1), _f32),
            jax.ShapeDtypeStruct((NP, 1), _f32),
        ],
    )(xp, W1, degt)


def _mid_body(a0_ref, a1_ref, hs_ref, dis_ref, b_ref, w_ref, o_ref):
    dis = dis_ref[...]
    z1 = jnp.maximum(
        (a0_ref[...] + a1_ref[...] + hs_ref[...]) * dis + b_ref[...], 0.0)
    o_ref[...] = jnp.dot(z1, w_ref[...], preferred_element_type=_f32) * dis


def _mid_call(a0, a1, hs1, dis, b1, W2):
    return pl.pallas_call(
        _mid_body,
        grid=(_GRID,),
        in_specs=[
            pl.BlockSpec((_BLK, H1), lambda i: (i, 0)),
            pl.BlockSpec((_BLK, H1), lambda i: (i, 0)),
            pl.BlockSpec((_BLK, H1), lambda i: (i, 0)),
            pl.BlockSpec((_BLK, 1), lambda i: (i, 0)),
            pl.BlockSpec((1, H1), lambda i: (0, 0)),
            pl.BlockSpec((H1, H2), lambda i: (0, 0)),
        ],
        out_specs=pl.BlockSpec((_BLK, H2), lambda i: (i, 0)),
        out_shape=jax.ShapeDtypeStruct((NP, H2), _f32),
    )(a0, a1, hs1, dis, b1, W2)


def _fin_body(a0_ref, a1_ref, hs_ref, dis_ref, b_ref, o_ref):
    o_ref[...] = ((a0_ref[...] + a1_ref[...] + hs_ref[...]) * dis_ref[...]
                  + b_ref[...]).astype(jnp.bfloat16)


def _fin_call(a0, a1, hs2, dis, b2):
    return pl.pallas_call(
        _fin_body,
        grid=(_GRID,),
        in_specs=[
            pl.BlockSpec((_BLK, H2), lambda i: (i, 0)),
            pl.BlockSpec((_BLK, H2), lambda i: (i, 0)),
            pl.BlockSpec((_BLK, H2), lambda i: (i, 0)),
            pl.BlockSpec((_BLK, 1), lambda i: (i, 0)),
            pl.BlockSpec((1, H2), lambda i: (0, 0)),
        ],
        out_specs=pl.BlockSpec((_BLK, H2), lambda i: (i, 0)),
        out_shape=jax.ShapeDtypeStruct((NP, H2), jnp.bfloat16),
    )(a0, a1, hs2, dis, b2)


def kernel(x, pos_edge_index, neg_edge_index, W1, b1, W2, b2):
    src = pos_edge_index[0].astype(_i32)
    dst = pos_edge_index[1].astype(_i32)
    xp = jnp.pad(x, ((0, NP - N), (0, 0)))

    degp = _deg_call(dst)
    degt = degp.T
    hs1, dis = _enc1_call(xp, W1, degt)
    acc1 = _scatter64(src, dst, hs1)
    hs2 = _mid_call(acc1[0], acc1[1], hs1, dis, b1.reshape(1, H1), W2)
    acc2 = _scatter32(src, dst, hs2)
    z2 = _fin_call(acc2[0], acc2[1], hs2, dis, b2.reshape(1, H2))

    s2 = jnp.concatenate([src, neg_edge_index[0].astype(_i32)])
    d2 = jnp.concatenate([dst, neg_edge_index[1].astype(_i32)])
    return _dec_call(s2, d2, z2)
